# baseline (speedup 1.0000x reference)
import functools

import jax
import jax.numpy as jnp
from jax import lax
from jax.experimental import pallas as pl
from jax.experimental.pallas import tpu as pltpu
from jax.experimental.pallas import tpu_sc as plsc

N, E, B, H = 10000, 320000, 256, 128

LN = 16
NC, NS = 2, 16
NW = NC * NS
EPW = E // NW
CHUNK = 80
NCH = EPW // CHUNK
NBUF = 4

_mesh = plsc.VectorSubcoreMesh(core_axis_name="c", subcore_axis_name="s")


@functools.partial(
    pl.kernel,
    out_type=(jax.ShapeDtypeStruct((NC, B, H), jnp.float32),
              jax.ShapeDtypeStruct((NC, B, H), jnp.float32),
              jax.ShapeDtypeStruct((NW, B * LN), jnp.float32)),
    mesh=_mesh,
    scratch_types=[
        pltpu.VMEM((N,), jnp.int32),
        pltpu.VMEM((EPW,), jnp.int32),
        pltpu.VMEM((NBUF, CHUNK, H), jnp.float32),
        pltpu.VMEM((NBUF, CHUNK), jnp.int32),
        pltpu.VMEM((CHUNK, H), jnp.float32),
        pltpu.VMEM((1, CHUNK), jnp.int32),
        pltpu.VMEM_SHARED((B, H), jnp.float32),
        pltpu.VMEM_SHARED((B, H), jnp.float32),
        pltpu.VMEM((B * LN,), jnp.float32),
        [pltpu.SemaphoreType.DMA] * NBUF,
        [pltpu.SemaphoreType.DMA] * NBUF,
    ],
    compiler_params=pltpu.CompilerParams(needs_layout_passes=False),
)
def _sc_pool(row_hbm, batch_hbm, edge_hbm, x_hbm, zsum_hbm,
             esums_hbm, xsums_hbm, cnts_hbm,
             batch_v, row_v, ebuf, idx_v, nbuf_v, nidx_v, acc_v, accx_v,
             cnt_v, dsem, ssem):
    sid = lax.axis_index("s")
    cid = lax.axis_index("c")
    wid = sid * NC + cid
    base = wid * EPW

    pltpu.sync_copy(batch_hbm, batch_v)
    pltpu.sync_copy(row_hbm.at[pl.ds(base, EPW)], row_v)

    @pl.when(sid == 0)
    def _zero_shared():
        pltpu.sync_copy(zsum_hbm, acc_v)
        pltpu.sync_copy(zsum_hbm, accx_v)

    zeros16 = jnp.zeros((LN,), jnp.float32)
    ones16 = jnp.ones((LN,), jnp.float32)
    lane_iota = lax.iota(jnp.int32, LN)

    def _zero_cnt(i, carry):
        for k in range(16):
            cnt_v[pl.ds(i * 256 + k * LN, LN)] = zeros16
        return carry
    lax.fori_loop(0, (B * LN) // 256, _zero_cnt, 0)

    plsc.subcore_barrier()

    def _chunk_src(c):
        return edge_hbm.at[pl.ds(base + c * CHUNK, CHUNK), :]

    def _fill_idx(c, s):
        for k in range(CHUNK // LN):
            r16 = row_v[pl.ds(c * CHUNK + k * LN, LN)]
            idx_v[s, pl.ds(k * LN, LN)] = plsc.load_gather(batch_v, [r16])

    def _scatter_desc(s):
        return pltpu.make_async_copy(ebuf.at[s], acc_v.at[idx_v.at[s]],
                                     ssem[s])

    pltpu.async_copy(_chunk_src(0), ebuf.at[0], dsem[0])
    pltpu.async_copy(_chunk_src(1), ebuf.at[1], dsem[1])

    def _turn(cc, carry):
        for s in range(NBUF):
            c = cc * NBUF + s

            @pl.when(c < NCH)
            def _process():
                pltpu.make_async_copy(_chunk_src(c), ebuf.at[s], dsem[s]).wait()
                _fill_idx(c, s)
                pltpu.async_copy(ebuf.at[s], acc_v.at[idx_v.at[s]], ssem[s],
                                 add=True)
                for k in range(CHUNK // LN):
                    sv = idx_v[s, pl.ds(k * LN, LN)]
                    tgt = sv * LN + lane_iota
                    plsc.addupdate_scatter(cnt_v, [tgt], ones16)

            sp = (s + 2) % NBUF

            @pl.when(c + 2 < NCH)
            def _prefetch():
                @pl.when(c >= 2)
                def _drain_prev():
                    _scatter_desc(sp).wait()
                pltpu.async_copy(_chunk_src(c + 2), ebuf.at[sp], dsem[sp])
        return carry
    lax.fori_loop(0, (NCH + NBUF - 1) // NBUF, _turn, 0)

    NCHX = N // CHUNK
    nj = (NCHX - wid + NW - 1) // NW

    def _node_chunk(i, carry):
        c = wid + i * NW
        pltpu.sync_copy(x_hbm.at[pl.ds(c * CHUNK, CHUNK), :], nbuf_v)
        for k in range(CHUNK // LN):
            nidx_v[0, pl.ds(k * LN, LN)] = batch_v[pl.ds(c * CHUNK + k * LN, LN)]
        pltpu.sync_copy(nbuf_v, accx_v.at[nidx_v.at[0]], add=True)
        return carry
    lax.fori_loop(0, nj, _node_chunk, 0)

    for cf in range(NCH - NBUF, NCH):
        _scatter_desc(cf % NBUF).wait()
    plsc.subcore_barrier()

    @pl.when(sid == 0)
    def _out_sums():
        pltpu.sync_copy(acc_v, esums_hbm.at[cid])
        pltpu.sync_copy(accx_v, xsums_hbm.at[cid])
    pltpu.sync_copy(cnt_v, cnts_hbm.at[wid])


def _tc_combine_body(ps_ref, px_ref, pc_ref, batch_ref, u_ref, w1_ref, b1_ref,
                     w2_ref, b2_ref, out_ref):
    dn = (((1,), (1,)), ((), ()))
    e_sum = jnp.sum(ps_ref[...], axis=0)
    cnt_col = jnp.sum(jnp.sum(pc_ref[...], axis=0), axis=1,
                      keepdims=True)
    e_mean = e_sum / jnp.maximum(cnt_col, 1.0)
    b_iota = jax.lax.broadcasted_iota(jnp.int32, (B, N), 0)
    hist_col = jnp.sum(jnp.equal(batch_ref[...], b_iota).astype(jnp.float32),
                       axis=1, keepdims=True)
    x_mean = jnp.sum(px_ref[...], axis=0) / jnp.maximum(hist_col, 1.0)
    cat = jnp.concatenate([u_ref[...], x_mean, e_mean], axis=1)
    h1 = jax.lax.dot_general(cat, w1_ref[...], dn,
                             preferred_element_type=jnp.float32) + b1_ref[...]
    h1 = jnp.maximum(h1, 0.0)
    out_ref[...] = jax.lax.dot_general(h1, w2_ref[...], dn,
                                       preferred_element_type=jnp.float32) + b2_ref[...]


def _tc_combine(esums, xsums, part_cnts, batch2, u, W1, b1r, W2, b2r):
    return pl.pallas_call(
        _tc_combine_body,
        grid=(1,),
        in_specs=[
            pl.BlockSpec((NC, B, H), lambda i: (0, 0, 0)),
            pl.BlockSpec((NC, B, H), lambda i: (0, 0, 0)),
            pl.BlockSpec((NW, B, LN), lambda i: (0, 0, 0)),
            pl.BlockSpec((1, N), lambda i: (0, 0)),
            pl.BlockSpec((B, H), lambda i: (0, 0)),
            pl.BlockSpec((H, 3 * H), lambda i: (0, 0)),
            pl.BlockSpec((1, H), lambda i: (0, 0)),
            pl.BlockSpec((H, H), lambda i: (0, 0)),
            pl.BlockSpec((1, H), lambda i: (0, 0)),
        ],
        out_specs=pl.BlockSpec((B, H), lambda i: (0, 0)),
        out_shape=jax.ShapeDtypeStruct((B, H), jnp.float32),
        compiler_params=pltpu.CompilerParams(
            dimension_semantics=("arbitrary",),
        ),
    )(esums, xsums, part_cnts, batch2, u, W1, b1r, W2, b2r)


def kernel(x, edge_index, edge_attr, u, batch, W1, b1, W2, b2):
    row = edge_index[0]
    zsum = jnp.zeros((B, H), jnp.float32)
    esums, xsums, part_cnts = _sc_pool(row, batch, edge_attr, x, zsum)
    return _tc_combine(esums, xsums, part_cnts.reshape(NW, B, LN),
                       batch.reshape(1, N), u, W1, b1.reshape(1, H),
                       W2, b2.reshape(1, H))
